# baseline (speedup 1.0000x reference)
import jax
import jax.numpy as jnp
from jax.experimental import pallas as pl
from jax.experimental.pallas import tpu as pltpu

HPAD = 8


def _dense_call(x, W, A1, A2):
    n, din = x.shape
    hc = W.shape[1]
    rb = 1000
    grid = (n // rb,)

    def body(x_ref, w_ref, a1_ref, a2_ref, h_ref, as_ref, ad_ref):
        hb = jnp.dot(x_ref[...], w_ref[...], preferred_element_type=jnp.float32)
        h_ref[...] = hb
        as_ref[...] = jnp.dot(hb, a1_ref[...], preferred_element_type=jnp.float32)
        ad_ref[...] = jnp.dot(hb, a2_ref[...], preferred_element_type=jnp.float32)

    return pl.pallas_call(
        body,
        grid=grid,
        in_specs=[
            pl.BlockSpec((rb, din), lambda g: (g, 0)),
            pl.BlockSpec((din, hc), lambda g: (0, 0)),
            pl.BlockSpec((hc, HPAD), lambda g: (0, 0)),
            pl.BlockSpec((hc, HPAD), lambda g: (0, 0)),
        ],
        out_specs=[
            pl.BlockSpec((rb, hc), lambda g: (g, 0)),
            pl.BlockSpec((rb, HPAD), lambda g: (g, 0)),
            pl.BlockSpec((rb, HPAD), lambda g: (g, 0)),
        ],
        out_shape=[
            jax.ShapeDtypeStruct((n, hc), jnp.float32),
            jax.ShapeDtypeStruct((n, HPAD), jnp.float32),
            jax.ShapeDtypeStruct((n, HPAD), jnp.float32),
        ],
    )(x, W, A1, A2)


def _edge_call(h, asrc, adst, src3, dst3, expmat, brow, relu):
    n, hc = h.shape
    g_num, _, eb = src3.shape

    def body(src_ref, dst_ref, h_ref, as_ref, ad_ref, exp_ref, b_ref,
             out_ref, m_ref, r_ref):
        g = pl.program_id(0)

        rb = 1000

        @pl.when(g == 0)
        def _init():
            def zstep(j, carry):
                sl = pl.ds(j * rb, rb)
                m_ref[sl, :] = jnp.full((rb, HPAD), -jnp.inf, jnp.float32)
                r_ref[sl, :] = jnp.zeros((rb, HPAD), jnp.float32)
                out_ref[sl, :] = jnp.zeros((rb, hc), jnp.float32)
                return carry
            jax.lax.fori_loop(0, n // rb, zstep, 0)

        def step(i, carry):
            s = src_ref[0, 0, i]
            d = dst_ref[0, 0, i]
            a = as_ref[pl.ds(s, 1), :] + ad_ref[pl.ds(d, 1), :]
            e = jnp.where(a >= 0, a, 0.2 * a)
            mold = m_ref[pl.ds(d, 1), :]
            mnew = jnp.maximum(mold, e)
            scale = jnp.exp(mold - mnew)
            p = jnp.exp(e - mnew)
            r_ref[pl.ds(d, 1), :] = r_ref[pl.ds(d, 1), :] * scale + p
            m_ref[pl.ds(d, 1), :] = mnew
            sc_row = jnp.dot(scale, exp_ref[...], preferred_element_type=jnp.float32)
            p_row = jnp.dot(p, exp_ref[...], preferred_element_type=jnp.float32)
            hrow = h_ref[pl.ds(s, 1), :]
            out_ref[pl.ds(d, 1), :] = (
                out_ref[pl.ds(d, 1), :] * sc_row + hrow * p_row)
            return carry

        jax.lax.fori_loop(0, eb, step, 0)

        @pl.when(g == g_num - 1)
        def _fin():
            def fstep(j, carry):
                sl = pl.ds(j * rb, rb)
                den = jnp.dot(r_ref[sl, :], exp_ref[...],
                              preferred_element_type=jnp.float32)
                safe = jnp.where(den > 0, den, 1.0)
                o = out_ref[sl, :] / safe + b_ref[...]
                if relu:
                    o = jnp.maximum(o, 0.0)
                out_ref[sl, :] = o
                return carry
            jax.lax.fori_loop(0, n // rb, fstep, 0)

    return pl.pallas_call(
        body,
        grid=(g_num,),
        in_specs=[
            pl.BlockSpec((1, 1, eb), lambda g: (g, 0, 0),
                         memory_space=pltpu.SMEM),
            pl.BlockSpec((1, 1, eb), lambda g: (g, 0, 0),
                         memory_space=pltpu.SMEM),
            pl.BlockSpec((n, hc), lambda g: (0, 0)),
            pl.BlockSpec((n, HPAD), lambda g: (0, 0)),
            pl.BlockSpec((n, HPAD), lambda g: (0, 0)),
            pl.BlockSpec((HPAD, hc), lambda g: (0, 0)),
            pl.BlockSpec((1, hc), lambda g: (0, 0)),
        ],
        out_specs=pl.BlockSpec((n, hc), lambda g: (0, 0)),
        out_shape=jax.ShapeDtypeStruct((n, hc), jnp.float32),
        scratch_shapes=[
            pltpu.VMEM((n, HPAD), jnp.float32),
            pltpu.VMEM((n, HPAD), jnp.float32),
        ],
        compiler_params=pltpu.CompilerParams(
            vmem_limit_bytes=100 * 1024 * 1024),
    )(src3, dst3, h, asrc, adst, expmat, brow)


def _gat_layer(x, src3, dst3, W, a_src, a_dst, b, relu):
    heads, c = a_src.shape
    hc = heads * c
    cols = jnp.arange(hc) // c
    expt = (cols[:, None] == jnp.arange(HPAD)[None, :]).astype(jnp.float32)
    A1 = a_src.reshape(-1, 1) * expt
    A2 = a_dst.reshape(-1, 1) * expt
    h, asrc, adst = _dense_call(x, W, A1, A2)
    return _edge_call(h, asrc, adst, src3, dst3, expt.T,
                      b.reshape(1, hc), relu)


def kernel(x, edge_index, W1, a_src1, a_dst1, b1, W2, a_src2, a_dst2, b2,
           W3, a_src3, a_dst3, b3):
    src = edge_index[0].astype(jnp.int32)
    dst = edge_index[1].astype(jnp.int32)
    e_total = src.shape[0]
    eb = 512
    g_num = e_total // eb
    src3 = src.reshape(g_num, 1, eb)
    dst3 = dst.reshape(g_num, 1, eb)
    h = _gat_layer(x, src3, dst3, W1, a_src1, a_dst1, b1, True)
    h = _gat_layer(h, src3, dst3, W2, a_src2, a_dst2, b2, True)
    return _gat_layer(h, src3, dst3, W3, a_src3, a_dst3, b3, False)


# speedup vs baseline: 1.2537x; 1.2537x over previous
import jax
import jax.numpy as jnp
from jax.experimental import pallas as pl
from jax.experimental.pallas import tpu as pltpu

HPAD = 8


def _dense_call(x, W, A1, A2):
    n, din = x.shape
    hc = W.shape[1]
    rb = 1000
    grid = (n // rb,)

    def body(x_ref, w_ref, a12_ref, h_ref, al_ref):
        hb = jnp.dot(x_ref[...], w_ref[...], preferred_element_type=jnp.float32)
        h_ref[...] = hb
        al_ref[...] = jnp.dot(hb, a12_ref[...],
                              preferred_element_type=jnp.float32)

    return pl.pallas_call(
        body,
        grid=grid,
        in_specs=[
            pl.BlockSpec((rb, din), lambda g: (g, 0)),
            pl.BlockSpec((din, hc), lambda g: (0, 0)),
            pl.BlockSpec((hc, 2 * HPAD), lambda g: (0, 0)),
        ],
        out_specs=[
            pl.BlockSpec((rb, hc), lambda g: (g, 0)),
            pl.BlockSpec((rb, 2 * HPAD), lambda g: (g, 0)),
        ],
        out_shape=[
            jax.ShapeDtypeStruct((n, hc), jnp.float32),
            jax.ShapeDtypeStruct((n, 2 * HPAD), jnp.float32),
        ],
    )(x, W, jnp.concatenate([A1, A2], axis=1))


def _edge_call(h, alog, src3, dst3, brow, relu):
    n, hc = h.shape
    g_num, _, eb = src3.shape
    reps = hc // HPAD
    unroll = 4

    def body(src_ref, dst_ref, h_ref, al_ref, b_ref, out_ref, mr_ref):
        g = pl.program_id(0)
        rb = 1000

        @pl.when(g == 0)
        def _init():
            def zstep(j, carry):
                sl = pl.ds(j * rb, rb)
                mr_ref[sl, 0:HPAD] = jnp.full((rb, HPAD), -jnp.inf,
                                              jnp.float32)
                mr_ref[sl, HPAD:2 * HPAD] = jnp.zeros((rb, HPAD),
                                                      jnp.float32)
                out_ref[sl, :] = jnp.zeros((rb, hc), jnp.float32)
                return carry
            jax.lax.fori_loop(0, n // rb, zstep, 0)

        def one_edge(i):
            s = src_ref[0, 0, i]
            d = dst_ref[0, 0, i]
            a = (al_ref[pl.ds(s, 1), 0:HPAD]
                 + al_ref[pl.ds(d, 1), HPAD:2 * HPAD])
            e = jnp.where(a >= 0, a, 0.2 * a)
            mold = mr_ref[pl.ds(d, 1), 0:HPAD]
            mnew = jnp.maximum(mold, e)
            scale = jnp.exp(mold - mnew)
            p = jnp.exp(e - mnew)
            mr_ref[pl.ds(d, 1), HPAD:2 * HPAD] = (
                mr_ref[pl.ds(d, 1), HPAD:2 * HPAD] * scale + p)
            mr_ref[pl.ds(d, 1), 0:HPAD] = mnew
            sc_row = pltpu.repeat(scale, reps, axis=1)
            p_row = pltpu.repeat(p, reps, axis=1)
            hrow = h_ref[pl.ds(s, 1), :]
            out_ref[pl.ds(d, 1), :] = (
                out_ref[pl.ds(d, 1), :] * sc_row + hrow * p_row)

        def step(i, carry):
            for u in range(unroll):
                one_edge(i * unroll + u)
            return carry

        jax.lax.fori_loop(0, eb // unroll, step, 0)

        @pl.when(g == g_num - 1)
        def _fin():
            def fstep(j, carry):
                sl = pl.ds(j * rb, rb)
                den = pltpu.repeat(mr_ref[sl, HPAD:2 * HPAD], reps, axis=1)
                safe = jnp.where(den > 0, den, 1.0)
                o = out_ref[sl, :] / safe + b_ref[...]
                if relu:
                    o = jnp.maximum(o, 0.0)
                out_ref[sl, :] = o
                return carry
            jax.lax.fori_loop(0, n // rb, fstep, 0)

    return pl.pallas_call(
        body,
        grid=(g_num,),
        in_specs=[
            pl.BlockSpec((1, 1, eb), lambda g: (g, 0, 0),
                         memory_space=pltpu.SMEM),
            pl.BlockSpec((1, 1, eb), lambda g: (g, 0, 0),
                         memory_space=pltpu.SMEM),
            pl.BlockSpec((n, hc), lambda g: (0, 0)),
            pl.BlockSpec((n, 2 * HPAD), lambda g: (0, 0)),
            pl.BlockSpec((1, hc), lambda g: (0, 0)),
        ],
        out_specs=pl.BlockSpec((n, hc), lambda g: (0, 0)),
        out_shape=jax.ShapeDtypeStruct((n, hc), jnp.float32),
        scratch_shapes=[
            pltpu.VMEM((n, 2 * HPAD), jnp.float32),
        ],
        compiler_params=pltpu.CompilerParams(
            vmem_limit_bytes=110 * 1024 * 1024),
    )(src3, dst3, h, alog, brow)


def _cols_cm(W, heads, c):
    din = W.shape[0]
    Wt = W.reshape(din, heads, c).transpose(0, 2, 1)
    pad = jnp.zeros((din, c, HPAD - heads), jnp.float32)
    return jnp.concatenate([Wt, pad], axis=-1).reshape(din, c * HPAD)


def _rows_cm(W, heads, c):
    dout = W.shape[1]
    Wr = W.reshape(heads, c, dout).transpose(1, 0, 2)
    pad = jnp.zeros((c, HPAD - heads, dout), jnp.float32)
    return jnp.concatenate([Wr, pad], axis=1).reshape(c * HPAD, dout)


def _vec_cm(v, heads, c):
    vt = v.reshape(heads, c).T
    pad = jnp.zeros((c, HPAD - heads), jnp.float32)
    return jnp.concatenate([vt, pad], axis=1).reshape(1, c * HPAD)


def _alpha_cm(a):
    heads, c = a.shape
    at = jnp.concatenate(
        [a.T, jnp.zeros((c, HPAD - heads), jnp.float32)], axis=1)
    eye_t = jnp.tile(jnp.eye(HPAD, dtype=jnp.float32), (c, 1))
    return at.reshape(-1, 1) * eye_t


def _gat_layer(x, src3, dst3, Wcm, a_src, a_dst, bcm, relu):
    h, alog = _dense_call(x, Wcm, _alpha_cm(a_src), _alpha_cm(a_dst))
    return _edge_call(h, alog, src3, dst3, bcm, relu)


def kernel(x, edge_index, W1, a_src1, a_dst1, b1, W2, a_src2, a_dst2, b2,
           W3, a_src3, a_dst3, b3):
    src = edge_index[0].astype(jnp.int32)
    dst = edge_index[1].astype(jnp.int32)
    e_total = src.shape[0]
    eb = 512
    g_num = e_total // eb
    src3 = src.reshape(g_num, 1, eb)
    dst3 = dst.reshape(g_num, 1, eb)

    W1cm = _cols_cm(W1, 7, 64)
    W2cm = _cols_cm(_rows_cm(W2, 7, 64), 6, 64)
    W3cm = _cols_cm(_rows_cm(W3, 6, 64), 6, 40)

    h = _gat_layer(x, src3, dst3, W1cm, a_src1, a_dst1,
                   _vec_cm(b1, 7, 64), True)
    h = _gat_layer(h, src3, dst3, W2cm, a_src2, a_dst2,
                   _vec_cm(b2, 6, 64), True)
    out = _gat_layer(h, src3, dst3, W3cm, a_src3, a_dst3,
                     _vec_cm(b3, 6, 40), False)
    j = jnp.arange(6 * 40)
    return out[:, (j % 40) * HPAD + j // 40]


# speedup vs baseline: 1.4273x; 1.1385x over previous
import jax
import jax.numpy as jnp
from jax.experimental import pallas as pl
from jax.experimental.pallas import tpu as pltpu

HPAD = 8


def _dense_call(x, W, A1, A2):
    n, din = x.shape
    hc = W.shape[1]
    rb = 1000
    grid = (n // rb,)

    def body(x_ref, w_ref, a12_ref, h_ref, al_ref):
        hb = jnp.dot(x_ref[...], w_ref[...], preferred_element_type=jnp.float32)
        h_ref[...] = hb
        al_ref[...] = jnp.dot(hb, a12_ref[...],
                              preferred_element_type=jnp.float32)

    return pl.pallas_call(
        body,
        grid=grid,
        in_specs=[
            pl.BlockSpec((rb, din), lambda g: (g, 0)),
            pl.BlockSpec((din, hc), lambda g: (0, 0)),
            pl.BlockSpec((hc, 2 * HPAD), lambda g: (0, 0)),
        ],
        out_specs=[
            pl.BlockSpec((rb, hc), lambda g: (g, 0)),
            pl.BlockSpec((rb, 2 * HPAD), lambda g: (g, 0)),
        ],
        out_shape=[
            jax.ShapeDtypeStruct((n, hc), jnp.float32),
            jax.ShapeDtypeStruct((n, 2 * HPAD), jnp.float32),
        ],
    )(x, W, jnp.concatenate([A1, A2], axis=1))


def _edge_call(h, alog, src3, dst3, brow, relu):
    n, hc = h.shape
    g_num, _, eb = src3.shape
    reps = hc // HPAD
    unroll = 8

    def body(src_ref, dst_ref, h_ref, al_ref, b_ref, out_ref, mr_ref):
        g = pl.program_id(0)
        rb = 1000

        @pl.when(g == 0)
        def _init():
            def zstep(j, carry):
                sl = pl.ds(j * rb, rb)
                mr_ref[sl, 0:HPAD] = jnp.full((rb, HPAD), -jnp.inf,
                                              jnp.float32)
                mr_ref[sl, HPAD:2 * HPAD] = jnp.zeros((rb, HPAD),
                                                      jnp.float32)
                out_ref[sl, :] = jnp.zeros((rb, hc), jnp.float32)
                return carry
            jax.lax.fori_loop(0, n // rb, zstep, 0)

        def one_edge(i):
            s = src_ref[0, 0, i]
            d = dst_ref[0, 0, i]
            a = (al_ref[pl.ds(s, 1), 0:HPAD]
                 + al_ref[pl.ds(d, 1), HPAD:2 * HPAD])
            e = jnp.where(a >= 0, a, 0.2 * a)
            mold = mr_ref[pl.ds(d, 1), 0:HPAD]
            mnew = jnp.maximum(mold, e)
            scale = jnp.exp(mold - mnew)
            p = jnp.exp(e - mnew)
            mr_ref[pl.ds(d, 1), HPAD:2 * HPAD] = (
                mr_ref[pl.ds(d, 1), HPAD:2 * HPAD] * scale + p)
            mr_ref[pl.ds(d, 1), 0:HPAD] = mnew
            sc_row = pltpu.repeat(scale, reps, axis=1)
            p_row = pltpu.repeat(p, reps, axis=1)
            hrow = h_ref[pl.ds(s, 1), :]
            out_ref[pl.ds(d, 1), :] = (
                out_ref[pl.ds(d, 1), :] * sc_row + hrow * p_row)

        def step(i, carry):
            for u in range(unroll):
                one_edge(i * unroll + u)
            return carry

        jax.lax.fori_loop(0, eb // unroll, step, 0)

        @pl.when(g == g_num - 1)
        def _fin():
            def fstep(j, carry):
                sl = pl.ds(j * rb, rb)
                den = pltpu.repeat(mr_ref[sl, HPAD:2 * HPAD], reps, axis=1)
                safe = jnp.where(den > 0, den, 1.0)
                o = out_ref[sl, :] / safe + b_ref[...]
                if relu:
                    o = jnp.maximum(o, 0.0)
                out_ref[sl, :] = o
                return carry
            jax.lax.fori_loop(0, n // rb, fstep, 0)

    return pl.pallas_call(
        body,
        grid=(g_num,),
        in_specs=[
            pl.BlockSpec((1, 1, eb), lambda g: (g, 0, 0),
                         memory_space=pltpu.SMEM),
            pl.BlockSpec((1, 1, eb), lambda g: (g, 0, 0),
                         memory_space=pltpu.SMEM),
            pl.BlockSpec((n, hc), lambda g: (0, 0)),
            pl.BlockSpec((n, 2 * HPAD), lambda g: (0, 0)),
            pl.BlockSpec((1, hc), lambda g: (0, 0)),
        ],
        out_specs=pl.BlockSpec((n, hc), lambda g: (0, 0)),
        out_shape=jax.ShapeDtypeStruct((n, hc), jnp.float32),
        scratch_shapes=[
            pltpu.VMEM((n, 2 * HPAD), jnp.float32),
        ],
        compiler_params=pltpu.CompilerParams(
            vmem_limit_bytes=110 * 1024 * 1024),
    )(src3, dst3, h, alog, brow)


def _cols_cm(W, heads, c):
    din = W.shape[0]
    Wt = W.reshape(din, heads, c).transpose(0, 2, 1)
    pad = jnp.zeros((din, c, HPAD - heads), jnp.float32)
    return jnp.concatenate([Wt, pad], axis=-1).reshape(din, c * HPAD)


def _rows_cm(W, heads, c):
    dout = W.shape[1]
    Wr = W.reshape(heads, c, dout).transpose(1, 0, 2)
    pad = jnp.zeros((c, HPAD - heads, dout), jnp.float32)
    return jnp.concatenate([Wr, pad], axis=1).reshape(c * HPAD, dout)


def _vec_cm(v, heads, c):
    vt = v.reshape(heads, c).T
    pad = jnp.zeros((c, HPAD - heads), jnp.float32)
    return jnp.concatenate([vt, pad], axis=1).reshape(1, c * HPAD)


def _alpha_cm(a):
    heads, c = a.shape
    at = jnp.concatenate(
        [a.T, jnp.zeros((c, HPAD - heads), jnp.float32)], axis=1)
    eye_t = jnp.tile(jnp.eye(HPAD, dtype=jnp.float32), (c, 1))
    return at.reshape(-1, 1) * eye_t


def _gat_layer(x, src3, dst3, Wcm, a_src, a_dst, bcm, relu):
    h, alog = _dense_call(x, Wcm, _alpha_cm(a_src), _alpha_cm(a_dst))
    return _edge_call(h, alog, src3, dst3, bcm, relu)


def kernel(x, edge_index, W1, a_src1, a_dst1, b1, W2, a_src2, a_dst2, b2,
           W3, a_src3, a_dst3, b3):
    src = edge_index[0].astype(jnp.int32)
    dst = edge_index[1].astype(jnp.int32)
    e_total = src.shape[0]
    eb = 512
    g_num = e_total // eb
    src3 = src.reshape(g_num, 1, eb)
    dst3 = dst.reshape(g_num, 1, eb)

    W1cm = _cols_cm(W1, 7, 64)
    W2cm = _cols_cm(_rows_cm(W2, 7, 64), 6, 64)
    W3cm = _cols_cm(_rows_cm(W3, 6, 64), 6, 40)

    h = _gat_layer(x, src3, dst3, W1cm, a_src1, a_dst1,
                   _vec_cm(b1, 7, 64), True)
    h = _gat_layer(h, src3, dst3, W2cm, a_src2, a_dst2,
                   _vec_cm(b2, 6, 64), True)
    out = _gat_layer(h, src3, dst3, W3cm, a_src3, a_dst3,
                     _vec_cm(b3, 6, 40), False)
    j = jnp.arange(6 * 40)
    return out[:, (j % 40) * HPAD + j // 40]


# speedup vs baseline: 1.5319x; 1.0733x over previous
import jax
import jax.numpy as jnp
from jax.experimental import pallas as pl
from jax.experimental.pallas import tpu as pltpu

HPAD = 8


def _dense_call(x, W, A1, A2):
    n, din = x.shape
    hc = W.shape[1]
    rb = 1000
    grid = (n // rb,)

    def body(x_ref, w_ref, a12_ref, h_ref, al_ref):
        hb = jnp.dot(x_ref[...], w_ref[...], preferred_element_type=jnp.float32)
        h_ref[...] = hb
        al_ref[...] = jnp.dot(hb, a12_ref[...],
                              preferred_element_type=jnp.float32)

    return pl.pallas_call(
        body,
        grid=grid,
        in_specs=[
            pl.BlockSpec((rb, din), lambda g: (g, 0)),
            pl.BlockSpec((din, hc), lambda g: (0, 0)),
            pl.BlockSpec((hc, 2 * HPAD), lambda g: (0, 0)),
        ],
        out_specs=[
            pl.BlockSpec((rb, hc), lambda g: (g, 0)),
            pl.BlockSpec((rb, 2 * HPAD), lambda g: (g, 0)),
        ],
        out_shape=[
            jax.ShapeDtypeStruct((n, hc), jnp.float32),
            jax.ShapeDtypeStruct((n, 2 * HPAD), jnp.float32),
        ],
    )(x, W, jnp.concatenate([A1, A2], axis=1))


def _edge_call(h, alog, src3, dst3, brow, relu):
    n, hc = h.shape
    g_num, _, eb = src3.shape
    reps = hc // HPAD
    unroll = 16

    def body(src_ref, dst_ref, h_ref, al_ref, b_ref, out_ref, mr_ref):
        g = pl.program_id(0)
        rb = 1000

        @pl.when(g == 0)
        def _init():
            def zstep(j, carry):
                sl = pl.ds(j * rb, rb)
                mr_ref[sl, 0:HPAD] = jnp.full((rb, HPAD), -jnp.inf,
                                              jnp.float32)
                mr_ref[sl, HPAD:2 * HPAD] = jnp.zeros((rb, HPAD),
                                                      jnp.float32)
                out_ref[sl, :] = jnp.zeros((rb, hc), jnp.float32)
                return carry
            jax.lax.fori_loop(0, n // rb, zstep, 0)

        def one_edge(i):
            s = src_ref[0, 0, i]
            d = dst_ref[0, 0, i]
            a = (al_ref[pl.ds(s, 1), 0:HPAD]
                 + al_ref[pl.ds(d, 1), HPAD:2 * HPAD])
            e = jnp.where(a >= 0, a, 0.2 * a)
            mold = mr_ref[pl.ds(d, 1), 0:HPAD]
            mnew = jnp.maximum(mold, e)
            scale = jnp.exp(mold - mnew)
            p = jnp.exp(e - mnew)
            mr_ref[pl.ds(d, 1), HPAD:2 * HPAD] = (
                mr_ref[pl.ds(d, 1), HPAD:2 * HPAD] * scale + p)
            mr_ref[pl.ds(d, 1), 0:HPAD] = mnew
            sc_row = pltpu.repeat(scale, reps, axis=1)
            p_row = pltpu.repeat(p, reps, axis=1)
            hrow = h_ref[pl.ds(s, 1), :]
            out_ref[pl.ds(d, 1), :] = (
                out_ref[pl.ds(d, 1), :] * sc_row + hrow * p_row)

        def step(i, carry):
            for u in range(unroll):
                one_edge(i * unroll + u)
            return carry

        jax.lax.fori_loop(0, eb // unroll, step, 0)

        @pl.when(g == g_num - 1)
        def _fin():
            def fstep(j, carry):
                sl = pl.ds(j * rb, rb)
                den = pltpu.repeat(mr_ref[sl, HPAD:2 * HPAD], reps, axis=1)
                safe = jnp.where(den > 0, den, 1.0)
                o = out_ref[sl, :] / safe + b_ref[...]
                if relu:
                    o = jnp.maximum(o, 0.0)
                out_ref[sl, :] = o
                return carry
            jax.lax.fori_loop(0, n // rb, fstep, 0)

    return pl.pallas_call(
        body,
        grid=(g_num,),
        in_specs=[
            pl.BlockSpec((1, 1, eb), lambda g: (g, 0, 0),
                         memory_space=pltpu.SMEM),
            pl.BlockSpec((1, 1, eb), lambda g: (g, 0, 0),
                         memory_space=pltpu.SMEM),
            pl.BlockSpec((n, hc), lambda g: (0, 0)),
            pl.BlockSpec((n, 2 * HPAD), lambda g: (0, 0)),
            pl.BlockSpec((1, hc), lambda g: (0, 0)),
        ],
        out_specs=pl.BlockSpec((n, hc), lambda g: (0, 0)),
        out_shape=jax.ShapeDtypeStruct((n, hc), jnp.float32),
        scratch_shapes=[
            pltpu.VMEM((n, 2 * HPAD), jnp.float32),
        ],
        compiler_params=pltpu.CompilerParams(
            vmem_limit_bytes=110 * 1024 * 1024),
    )(src3, dst3, h, alog, brow)


def _cols_cm(W, heads, c):
    din = W.shape[0]
    Wt = W.reshape(din, heads, c).transpose(0, 2, 1)
    pad = jnp.zeros((din, c, HPAD - heads), jnp.float32)
    return jnp.concatenate([Wt, pad], axis=-1).reshape(din, c * HPAD)


def _rows_cm(W, heads, c):
    dout = W.shape[1]
    Wr = W.reshape(heads, c, dout).transpose(1, 0, 2)
    pad = jnp.zeros((c, HPAD - heads, dout), jnp.float32)
    return jnp.concatenate([Wr, pad], axis=1).reshape(c * HPAD, dout)


def _vec_cm(v, heads, c):
    vt = v.reshape(heads, c).T
    pad = jnp.zeros((c, HPAD - heads), jnp.float32)
    return jnp.concatenate([vt, pad], axis=1).reshape(1, c * HPAD)


def _alpha_cm(a):
    heads, c = a.shape
    at = jnp.concatenate(
        [a.T, jnp.zeros((c, HPAD - heads), jnp.float32)], axis=1)
    eye_t = jnp.tile(jnp.eye(HPAD, dtype=jnp.float32), (c, 1))
    return at.reshape(-1, 1) * eye_t


def _gat_layer(x, src3, dst3, Wcm, a_src, a_dst, bcm, relu):
    h, alog = _dense_call(x, Wcm, _alpha_cm(a_src), _alpha_cm(a_dst))
    return _edge_call(h, alog, src3, dst3, bcm, relu)


def kernel(x, edge_index, W1, a_src1, a_dst1, b1, W2, a_src2, a_dst2, b2,
           W3, a_src3, a_dst3, b3):
    src = edge_index[0].astype(jnp.int32)
    dst = edge_index[1].astype(jnp.int32)
    e_total = src.shape[0]
    eb = 512
    g_num = e_total // eb
    src3 = src.reshape(g_num, 1, eb)
    dst3 = dst.reshape(g_num, 1, eb)

    W1cm = _cols_cm(W1, 7, 64)
    W2cm = _cols_cm(_rows_cm(W2, 7, 64), 6, 64)
    W3cm = _cols_cm(_rows_cm(W3, 6, 64), 6, 40)

    h = _gat_layer(x, src3, dst3, W1cm, a_src1, a_dst1,
                   _vec_cm(b1, 7, 64), True)
    h = _gat_layer(h, src3, dst3, W2cm, a_src2, a_dst2,
                   _vec_cm(b2, 6, 64), True)
    out = _gat_layer(h, src3, dst3, W3cm, a_src3, a_dst3,
                     _vec_cm(b3, 6, 40), False)
    j = jnp.arange(6 * 40)
    return out[:, (j % 40) * HPAD + j // 40]
